# baseline (speedup 1.0000x reference)
import functools

import jax
import jax.numpy as jnp
from jax import lax
from jax.experimental import pallas as pl
from jax.experimental.pallas import tpu as pltpu
from jax.experimental.pallas import tpu_sc as plsc

_N1, _N2, _N3 = 10000, 20000, 40000
_HID = 192
_CNN_SPECS = [(16, 3, 1), (16, 3, 1), (32, 3, 2), (32, 3, 1), (32, 3, 1),
              (64, 3, 2), (64, 3, 1), (64, 3, 1), (128, 3, 2), (128, 3, 1),
              (128, 3, 1), (256, 5, 2), (256, 3, 1), (256, 3, 1),
              (512, 5, 2), (512, 3, 1), (512, 3, 1), (512, 3, 1)]
_FEAT_TAPS = (7, 10, 13, 17)

_BN = 512
_EB = 128
_NT = 32


def _round_up(v, m):
    return (v + m - 1) // m * m


def _mm2_body(o, x_ref, w_ref, b_ref, d_ref, p_ref):
    y = jnp.dot(x_ref[...], w_ref[...], preferred_element_type=jnp.float32)
    d_ref[...] = y[:, :o] + b_ref[...]
    p_ref[...] = y[:, o:]


def _mm2(x, wc, b, o):
    npad, k = x.shape
    out = jax.ShapeDtypeStruct((npad, o), jnp.float32)
    return pl.pallas_call(
        functools.partial(_mm2_body, o),
        grid=(npad // _BN,),
        in_specs=[
            pl.BlockSpec((_BN, k), lambda i: (i, 0)),
            pl.BlockSpec((k, 2 * o), lambda i: (0, 0)),
            pl.BlockSpec((1, o), lambda i: (0, 0)),
        ],
        out_specs=[pl.BlockSpec((_BN, o), lambda i: (i, 0))] * 2,
        out_shape=[out, out],
    )(x, wc, b)


def _finish_body(relu, avg, d_ref, a_ref, s_ref, o_ref):
    v = d_ref[...] + a_ref[...]
    if relu:
        v = jnp.maximum(v, 0.0)
    if avg:
        v = (v + s_ref[...]) * 0.5
    o_ref[...] = v


def _finish(dense, agg, skip, relu):
    npad, o = dense.shape
    avg = skip is not None
    if skip is None:
        skip = dense
    return pl.pallas_call(
        functools.partial(_finish_body, relu, avg),
        grid=(npad // _BN,),
        in_specs=[pl.BlockSpec((_BN, o), lambda i: (i, 0))] * 3,
        out_specs=pl.BlockSpec((_BN, o), lambda i: (i, 0)),
        out_shape=jax.ShapeDtypeStruct((npad, o), jnp.float32),
    )(dense, agg, skip)


_MESH = plsc.VectorSubcoreMesh(core_axis_name="c", subcore_axis_name="s")


@functools.lru_cache(maxsize=None)
def _make_bin_kernel(e_pad, nch, chunk):
    cap = e_pad // _NT
    cap_pad = _round_up(cap, 2 * _EB)
    sb = 480
    nfull, rem = cap // sb, cap % sb

    @functools.partial(
        pl.kernel,
        out_type=(
            jax.ShapeDtypeStruct((nch, _NT, cap_pad), jnp.int32),
            jax.ShapeDtypeStruct((nch, _NT, cap_pad), jnp.float32),
            jax.ShapeDtypeStruct((_NT, 16), jnp.int32),
        ),
        mesh=_MESH,
        compiler_params=pltpu.CompilerParams(use_tc_tiling_on_sc=False),
        scratch_types=[
            pltpu.VMEM((sb,), jnp.int32),
            pltpu.VMEM((sb,), jnp.int32),
            pltpu.VMEM((sb,), jnp.float32),
            pltpu.VMEM((nch * cap_pad,), jnp.int32),
            pltpu.VMEM((nch * cap_pad,), jnp.float32),
            pltpu.VMEM((16,), jnp.int32),
            pltpu.SMEM((8,), jnp.int32),
        ],
    )
    def bin_kernel(src_hbm, dst_hbm, ew_hbm, keyb_hbm, ewb_hbm, cnt_hbm,
                   src_v, dst_v, ew_v, keybuf, ewbuf, cnt_v, cur_s):
        wid = lax.axis_index("c") * 16 + lax.axis_index("s")
        base = wid * cap

        zi = jnp.zeros((16,), jnp.int32)
        zf = jnp.zeros((16,), jnp.float32)

        def zero_step(j, _):
            keybuf[pl.ds(j * 16, 16)] = zi
            ewbuf[pl.ds(j * 16, 16)] = zf
            return 0
        lax.fori_loop(0, nch * cap_pad // 16, zero_step, 0)
        for c in range(nch):
            cur_s[c] = jnp.int32(c * cap_pad)

        def step(i, _):
            s16 = src_v[pl.ds(i * 16, 16)]
            d16 = dst_v[pl.ds(i * 16, 16)]
            w16 = ew_v[pl.ds(i * 16, 16)]
            b16 = jnp.zeros((16,), jnp.int32)
            for c in range(1, nch):
                b16 = b16 + jnp.where(d16 >= c * chunk, 1, 0)
            dloc16 = d16 - b16 * chunk
            key16 = s16 | (dloc16 << 16)
            for l in range(16):
                bl = b16[l]
                cur = cur_s[bl]
                keybuf[pl.ds(cur, 16)] = jnp.full((16,), key16[l], jnp.int32)
                ewbuf[pl.ds(cur, 16)] = jnp.full((16,), w16[l], jnp.float32)
                cur_s[bl] = cur + 1
            return 0

        def stage(j, _):
            off = base + j * sb
            pltpu.sync_copy(src_hbm.at[pl.ds(off, sb)], src_v)
            pltpu.sync_copy(dst_hbm.at[pl.ds(off, sb)], dst_v)
            pltpu.sync_copy(ew_hbm.at[pl.ds(off, sb)], ew_v)
            lax.fori_loop(0, sb // 16, step, 0)
            return 0
        lax.fori_loop(0, nfull, stage, 0)
        if rem:
            off = base + nfull * sb
            pltpu.sync_copy(src_hbm.at[pl.ds(off, rem)], src_v.at[pl.ds(0, rem)])
            pltpu.sync_copy(dst_hbm.at[pl.ds(off, rem)], dst_v.at[pl.ds(0, rem)])
            pltpu.sync_copy(ew_hbm.at[pl.ds(off, rem)], ew_v.at[pl.ds(0, rem)])
            lax.fori_loop(0, rem // 16, step, 0)

        for c in range(nch):
            end = cur_s[c]
            keybuf[pl.ds(end, 16)] = zi
            ewbuf[pl.ds(end, 16)] = zf

        for c in range(nch):
            pltpu.sync_copy(keybuf.at[pl.ds(c * cap_pad, cap_pad)],
                            keyb_hbm.at[c, wid])
            pltpu.sync_copy(ewbuf.at[pl.ds(c * cap_pad, cap_pad)],
                            ewb_hbm.at[c, wid])
        lanes = lax.iota(jnp.int32, 16)
        cv = jnp.zeros((16,), jnp.int32)
        for c in range(nch):
            cv = jnp.where(lanes == c,
                           jnp.full((16,), cur_s[c] - c * cap_pad, jnp.int32),
                           cv)
        cnt_v[pl.ds(0, 16)] = cv
        pltpu.sync_copy(cnt_v, cnt_hbm.at[wid])

    return bin_kernel


@functools.lru_cache(maxsize=None)
def _make_seg_kernel(e_pad, nch, chunk, npad, w):
    cap = e_pad // _NT
    cap_pad = _round_up(cap, 2 * _EB)
    npc = nch // 2
    rps = chunk // 16
    zr = 64
    nw = w // 16

    @functools.partial(
        pl.kernel,
        out_type=jax.ShapeDtypeStruct((npad, w), jnp.float32),
        mesh=_MESH,
        compiler_params=pltpu.CompilerParams(use_tc_tiling_on_sc=False),
        scratch_types=[
            pltpu.VMEM((_EB,), jnp.int32),
            pltpu.VMEM((_EB,), jnp.float32),
            pltpu.VMEM((_EB,), jnp.int32),
            pltpu.VMEM((_EB,), jnp.int32),
            pltpu.VMEM((_EB,), jnp.int32),
            pltpu.VMEM((_EB,), jnp.float32),
            pltpu.VMEM((_EB,), jnp.int32),
            pltpu.VMEM((_EB,), jnp.int32),
            pltpu.VMEM((_EB, w), jnp.float32),
            pltpu.VMEM((_EB, w), jnp.float32),
            pltpu.VMEM((zr, w), jnp.float32),
            pltpu.VMEM((16,), jnp.int32),
            pltpu.VMEM_SHARED((chunk, w), jnp.float32),
            pltpu.SemaphoreType.DMA,
            pltpu.SemaphoreType.DMA,
        ],
    )
    def seg_kernel(pre_hbm, keyb_hbm, ewb_hbm, cnt_hbm, agg_hbm,
                   key0, ew0, idx0, dloc0, key1, ew1, idx1, dloc1,
                   rows0, rows1, zer_v, cnt_v, shared, sem0, sem1):
        core = lax.axis_index("c")
        sub = lax.axis_index("s")
        meta = ((key0, ew0, idx0, dloc0), (key1, ew1, idx1, dloc1))
        rows = (rows0, rows1)
        sems = (sem0, sem1)

        zf = jnp.zeros((16,), jnp.float32)

        def zzero(t, _):
            row = t // nw
            col = t % nw
            zer_v[row, pl.ds(col * 16, 16)] = zf
            return 0
        lax.fori_loop(0, zr * nw, zzero, 0)

        for p in range(npc):
            b_idx = core * npc + p
            for z in range(rps // zr):
                pltpu.sync_copy(zer_v, shared.at[pl.ds(sub * rps + z * zr,
                                                       zr)])
            plsc.subcore_barrier()
            for rr in range(2):
                r = sub * 2 + rr
                pltpu.sync_copy(cnt_hbm.at[r], cnt_v)
                cv = cnt_v[pl.ds(0, 16)]
                cnt = jnp.where(core == 0, cv[p], cv[npc + p])
                nb = (cnt + (_EB - 1)) // _EB
                niter = (jnp.maximum(nb, 1) + 1) // 2
                last = niter * 2 - 1

                def load_meta(k, m):
                    key_v, ew_v, idx_v, dloc_v = meta[m]
                    off = jnp.minimum(k, last) * _EB
                    pltpu.sync_copy(keyb_hbm.at[b_idx, r, pl.ds(off, _EB)],
                                    key_v)
                    pltpu.sync_copy(ewb_hbm.at[b_idx, r, pl.ds(off, _EB)],
                                    ew_v)

                    def unpack(j, _):
                        k16 = key_v[pl.ds(j * 16, 16)]
                        idx_v[pl.ds(j * 16, 16)] = k16 & 0xFFFF
                        dloc_v[pl.ds(j * 16, 16)] = k16 >> 16
                        return 0
                    lax.fori_loop(0, _EB // 16, unpack, 0)

                def fire(m):
                    pltpu.async_copy(pre_hbm.at[meta[m][2]], rows[m],
                                     sems[m])

                def finish(m):
                    ew_v, dloc_v = meta[m][1], meta[m][3]
                    rows_v = rows[m]
                    pltpu.make_async_copy(pre_hbm.at[meta[m][2]], rows_v,
                                          sems[m]).wait()

                    def scale(g, _):
                        w16 = ew_v[pl.ds(g * 16, 16)]
                        for l in range(16):
                            e = g * 16 + l
                            wv = jnp.full((16,), w16[l], jnp.float32)
                            for j in range(nw):
                                rows_v[e, pl.ds(j * 16, 16)] = (
                                    rows_v[e, pl.ds(j * 16, 16)] * wv)
                        return 0
                    lax.fori_loop(0, _EB // 16, scale, 0)

                    pltpu.sync_copy(rows_v, shared.at[dloc_v], add=True)

                load_meta(jnp.int32(0), 0)
                fire(0)

                def pair(j, _):
                    load_meta(2 * j + 1, 1)
                    fire(1)
                    finish(0)
                    load_meta(2 * j + 2, 0)
                    fire(0)
                    finish(1)
                    return 0
                lax.fori_loop(0, niter, pair, 0)
                pltpu.make_async_copy(pre_hbm.at[meta[0][2]], rows[0],
                                      sems[0]).wait()
            plsc.subcore_barrier()
            pltpu.sync_copy(
                shared.at[pl.ds(sub * rps, rps)],
                agg_hbm.at[pl.ds(b_idx * chunk + sub * rps, rps)])
            plsc.subcore_barrier()

    return seg_kernel


def _gcn_block(x, params, binned, n_pad, e_pad, nch, chunk):
    keyb, ewb, cnts = binned
    acts = [x]
    L = len(params)
    for i, (w0, w1, b) in enumerate(params):
        xi = acts[-1]
        o = w0.shape[1]
        op = max(o, 128)
        osc = o if o >= 16 else 16
        wc = jnp.zeros((xi.shape[1], 2 * op), jnp.float32)
        wc = wc.at[:, :o].set(w0).at[:, op:op + o].set(w1)
        bc = jnp.zeros((1, op), jnp.float32).at[0, :o].set(b[:o])
        dense, pre = _mm2(xi, wc, bc, op)
        pre_sc = pre if osc == op else pre[:, :osc]
        agg = _make_seg_kernel(e_pad, nch, chunk, n_pad, osc)(
            pre_sc, keyb, ewb, cnts)
        if osc != op:
            agg = jnp.pad(agg, ((0, 0), (0, op - osc)))
        h = _finish(dense, agg, acts[-2] if i in (2, 4, 6, 8, 10, 12) else None,
                    relu=(i < L - 1))
        acts.append(h)
    return acts[-1], acts[-2]


def _bin_edges(ei, ew, nch, chunk):
    e = ew.shape[0]
    e_pad = _round_up(e, 512)
    pad = e_pad - e
    src = jnp.pad(ei[0], (0, pad))
    dst = jnp.pad(ei[1], (0, pad))
    eww = jnp.pad(ew, (0, pad))
    return _make_bin_kernel(e_pad, nch, chunk)(src, dst, eww), e_pad


def _cnn(img, params):
    x = img
    feats = []
    for i, (w, b) in enumerate(params):
        s = _CNN_SPECS[i][2]
        x = jax.lax.conv_general_dilated(
            x, w, (s, s), "SAME", dimension_numbers=("NHWC", "HWIO", "NHWC"))
        x = jax.nn.relu(x + b)
        if i in _FEAT_TAPS:
            feats.append(x[0])
    return feats


def _bilinear(im, y, x):
    h, w = im.shape[0], im.shape[1]
    y = jnp.clip(y, 0.0, h - 1.0)
    x = jnp.clip(x, 0.0, w - 1.0)
    y0 = jnp.floor(y).astype(jnp.int32)
    x0 = jnp.floor(x).astype(jnp.int32)
    y1 = jnp.minimum(y0 + 1, h - 1)
    x1 = jnp.minimum(x0 + 1, w - 1)
    wy = (y - y0)[:, None]
    wx = (x - x0)[:, None]
    top = im[y0, x0] * (1.0 - wx) + im[y0, x1] * wx
    bot = im[y1, x0] * (1.0 - wx) + im[y1, x1] * wx
    return top * (1.0 - wy) + bot * wy


def _projection(coords, cameras, feats):
    xc = coords @ cameras[:, :3].T + cameras[:, 3]
    z = xc[:, 2]
    z = jnp.where(jnp.abs(z) < 1e-2, 1e-2, z)
    hh = jnp.clip(250.0 * (-xc[:, 1] / z) + 112.0, 0.0, 223.0)
    ww = jnp.clip(250.0 * (xc[:, 0] / z) + 112.0, 0.0, 223.0)
    outs = [coords]
    for im in feats:
        s = im.shape[0]
        outs.append(_bilinear(im, hh * (s / 224.0), ww * (s / 224.0)))
    return jnp.concatenate(outs, axis=1)


def _pool(x, pidx):
    new = 0.5 * (x[pidx[:, 0]] + x[pidx[:, 1]])
    return jnp.concatenate([x, new], axis=0)


def _pad_rows(x, n_pad):
    return jnp.zeros((n_pad, x.shape[1]), x.dtype).at[:x.shape[0]].set(x)


def kernel(img_all_view, cameras, features, edge_index1, edge_w1, edge_index2,
           edge_w2, edge_index3, edge_w3, pool_idx1, pool_idx2, cnn_params,
           gcn_params):
    n1p, n2p, n3p = 10240, 20480, 40960
    feats = _cnn(img_all_view, cnn_params)
    p1, p2, p3 = gcn_params

    bin1, e1p = _bin_edges(edge_index1, edge_w1, 2, 5120)
    bin2, e2p = _bin_edges(edge_index2, edge_w2, 4, 5120)
    bin3, e3p = _bin_edges(edge_index3, edge_w3, 8, 5120)

    x1 = _pad_rows(_projection(features, cameras, feats), n1p)
    o1, h1 = _gcn_block(x1, p1, bin1, n1p, e1p, 2, 5120)
    out1 = o1[:_N1, :3]
    hid1 = h1[:_N1]

    x2 = jnp.concatenate([_projection(out1, cameras, feats), hid1], axis=1)
    x2 = _pad_rows(_pool(x2, pool_idx1), n2p)
    o2, h2 = _gcn_block(x2, p2, bin2, n2p, e2p, 4, 5120)
    out2 = o2[:_N2, :3]
    hid2 = h2[:_N2]

    x3 = jnp.concatenate([_projection(out2, cameras, feats), hid2], axis=1)
    x3 = _pad_rows(_pool(x3, pool_idx2), n3p)
    o3, _ = _gcn_block(x3, p3, bin3, n3p, e3p, 8, 5120)
    out3 = o3[:_N3, :3]

    out1_2 = _pool(out1, pool_idx1)
    out2_2 = _pool(out2, pool_idx2)
    return (out1, out2, out3, out1_2, out2_2)


# speedup vs baseline: 1.5356x; 1.5356x over previous
import functools

import jax
import jax.numpy as jnp
from jax import lax
from jax.experimental import pallas as pl
from jax.experimental.pallas import tpu as pltpu
from jax.experimental.pallas import tpu_sc as plsc

_N1, _N2, _N3 = 10000, 20000, 40000
_HID = 192
_CNN_SPECS = [(16, 3, 1), (16, 3, 1), (32, 3, 2), (32, 3, 1), (32, 3, 1),
              (64, 3, 2), (64, 3, 1), (64, 3, 1), (128, 3, 2), (128, 3, 1),
              (128, 3, 1), (256, 5, 2), (256, 3, 1), (256, 3, 1),
              (512, 5, 2), (512, 3, 1), (512, 3, 1), (512, 3, 1)]
_FEAT_TAPS = (7, 10, 13, 17)

_BN = 512
_EB = 128
_NT = 32


def _round_up(v, m):
    return (v + m - 1) // m * m


def _mm2_body(o, x_ref, w_ref, b_ref, d_ref, p_ref):
    y = jnp.dot(x_ref[...], w_ref[...], preferred_element_type=jnp.float32)
    d_ref[...] = y[:, :o] + b_ref[...]
    p_ref[...] = y[:, o:]


def _mm2(x, wc, b, o):
    npad, k = x.shape
    out = jax.ShapeDtypeStruct((npad, o), jnp.float32)
    return pl.pallas_call(
        functools.partial(_mm2_body, o),
        grid=(npad // _BN,),
        in_specs=[
            pl.BlockSpec((_BN, k), lambda i: (i, 0)),
            pl.BlockSpec((k, 2 * o), lambda i: (0, 0)),
            pl.BlockSpec((1, o), lambda i: (0, 0)),
        ],
        out_specs=[pl.BlockSpec((_BN, o), lambda i: (i, 0))] * 2,
        out_shape=[out, out],
    )(x, wc, b)


def _finish_body(relu, avg, d_ref, a_ref, s_ref, o_ref):
    v = d_ref[...] + a_ref[...]
    if relu:
        v = jnp.maximum(v, 0.0)
    if avg:
        v = (v + s_ref[...]) * 0.5
    o_ref[...] = v


def _finish(dense, agg, skip, relu):
    npad, o = dense.shape
    avg = skip is not None
    if skip is None:
        skip = dense
    return pl.pallas_call(
        functools.partial(_finish_body, relu, avg),
        grid=(npad // _BN,),
        in_specs=[pl.BlockSpec((_BN, o), lambda i: (i, 0))] * 3,
        out_specs=pl.BlockSpec((_BN, o), lambda i: (i, 0)),
        out_shape=jax.ShapeDtypeStruct((npad, o), jnp.float32),
    )(dense, agg, skip)


_MESH = plsc.VectorSubcoreMesh(core_axis_name="c", subcore_axis_name="s")


@functools.lru_cache(maxsize=None)
def _make_bin_kernel(e_pad, nch, chunk):
    cap = e_pad // _NT
    cap_pad = _round_up(cap, _EB)
    sb = 480
    nfull, rem = cap // sb, cap % sb

    @functools.partial(
        pl.kernel,
        out_type=(
            jax.ShapeDtypeStruct((nch, _NT, cap_pad), jnp.int32),
            jax.ShapeDtypeStruct((nch, _NT, cap_pad), jnp.float32),
            jax.ShapeDtypeStruct((_NT, 16), jnp.int32),
        ),
        mesh=_MESH,
        compiler_params=pltpu.CompilerParams(use_tc_tiling_on_sc=False),
        scratch_types=[
            pltpu.VMEM((sb,), jnp.int32),
            pltpu.VMEM((sb,), jnp.int32),
            pltpu.VMEM((sb,), jnp.float32),
            pltpu.VMEM((nch * cap_pad,), jnp.int32),
            pltpu.VMEM((nch * cap_pad,), jnp.float32),
            pltpu.VMEM((16,), jnp.int32),
            pltpu.SMEM((8,), jnp.int32),
        ],
    )
    def bin_kernel(src_hbm, dst_hbm, ew_hbm, keyb_hbm, ewb_hbm, cnt_hbm,
                   src_v, dst_v, ew_v, keybuf, ewbuf, cnt_v, cur_s):
        wid = lax.axis_index("c") * 16 + lax.axis_index("s")
        base = wid * cap

        zi = jnp.zeros((16,), jnp.int32)
        zf = jnp.zeros((16,), jnp.float32)

        def zero_step(j, _):
            keybuf[pl.ds(j * 16, 16)] = zi
            ewbuf[pl.ds(j * 16, 16)] = zf
            return 0
        lax.fori_loop(0, nch * cap_pad // 16, zero_step, 0)
        for c in range(nch):
            cur_s[c] = jnp.int32(c * cap_pad)

        def step(i, _):
            s16 = src_v[pl.ds(i * 16, 16)]
            d16 = dst_v[pl.ds(i * 16, 16)]
            w16 = ew_v[pl.ds(i * 16, 16)]
            b16 = jnp.zeros((16,), jnp.int32)
            for c in range(1, nch):
                b16 = b16 + jnp.where(d16 >= c * chunk, 1, 0)
            dloc16 = d16 - b16 * chunk
            key16 = s16 | (dloc16 << 16)
            for l in range(16):
                bl = b16[l]
                cur = cur_s[bl]
                keybuf[pl.ds(cur, 16)] = jnp.full((16,), key16[l], jnp.int32)
                ewbuf[pl.ds(cur, 16)] = jnp.full((16,), w16[l], jnp.float32)
                cur_s[bl] = cur + 1
            return 0

        def stage(j, _):
            off = base + j * sb
            pltpu.sync_copy(src_hbm.at[pl.ds(off, sb)], src_v)
            pltpu.sync_copy(dst_hbm.at[pl.ds(off, sb)], dst_v)
            pltpu.sync_copy(ew_hbm.at[pl.ds(off, sb)], ew_v)
            lax.fori_loop(0, sb // 16, step, 0)
            return 0
        lax.fori_loop(0, nfull, stage, 0)
        if rem:
            off = base + nfull * sb
            pltpu.sync_copy(src_hbm.at[pl.ds(off, rem)], src_v.at[pl.ds(0, rem)])
            pltpu.sync_copy(dst_hbm.at[pl.ds(off, rem)], dst_v.at[pl.ds(0, rem)])
            pltpu.sync_copy(ew_hbm.at[pl.ds(off, rem)], ew_v.at[pl.ds(0, rem)])
            lax.fori_loop(0, rem // 16, step, 0)

        for c in range(nch):
            end = cur_s[c]
            keybuf[pl.ds(end, 16)] = zi
            ewbuf[pl.ds(end, 16)] = zf

        for c in range(nch):
            pltpu.sync_copy(keybuf.at[pl.ds(c * cap_pad, cap_pad)],
                            keyb_hbm.at[c, wid])
            pltpu.sync_copy(ewbuf.at[pl.ds(c * cap_pad, cap_pad)],
                            ewb_hbm.at[c, wid])
        lanes = lax.iota(jnp.int32, 16)
        cv = jnp.zeros((16,), jnp.int32)
        for c in range(nch):
            cv = jnp.where(lanes == c,
                           jnp.full((16,), cur_s[c] - c * cap_pad, jnp.int32),
                           cv)
        cnt_v[pl.ds(0, 16)] = cv
        pltpu.sync_copy(cnt_v, cnt_hbm.at[wid])

    return bin_kernel


@functools.lru_cache(maxsize=None)
def _make_seg_kernel(e_pad, nch, chunk, npad, w):
    cap = e_pad // _NT
    cap_pad = _round_up(cap, _EB)
    npc = nch // 2
    rps = chunk // 16
    zr = 64
    nw = w // 16

    @functools.partial(
        pl.kernel,
        out_type=jax.ShapeDtypeStruct((npad, w), jnp.float32),
        mesh=_MESH,
        compiler_params=pltpu.CompilerParams(use_tc_tiling_on_sc=False),
        scratch_types=[
            pltpu.VMEM((_EB,), jnp.int32),
            pltpu.VMEM((_EB,), jnp.float32),
            pltpu.VMEM((_EB,), jnp.int32),
            pltpu.VMEM((_EB,), jnp.int32),
            pltpu.VMEM((_EB, w), jnp.float32),
            pltpu.VMEM((zr, w), jnp.float32),
            pltpu.VMEM((16,), jnp.int32),
            pltpu.VMEM_SHARED((chunk, w), jnp.float32),
            pltpu.SemaphoreType.DMA,
        ],
    )
    def seg_kernel(pre_hbm, keyb_hbm, ewb_hbm, cnt_hbm, agg_hbm,
                   key_v, ew_v, idx_v, dloc_v, rows_v, zer_v, cnt_v,
                   shared, sem):
        core = lax.axis_index("c")
        sub = lax.axis_index("s")

        zf = jnp.zeros((16,), jnp.float32)

        def zzero(t, _):
            row = t // nw
            col = t % nw
            zer_v[row, pl.ds(col * 16, 16)] = zf
            return 0
        lax.fori_loop(0, zr * nw, zzero, 0)

        for p in range(npc):
            b_idx = core * npc + p
            for z in range(rps // zr):
                pltpu.sync_copy(zer_v, shared.at[pl.ds(sub * rps + z * zr,
                                                       zr)])
            plsc.subcore_barrier()
            for rr in range(2):
                r = sub * 2 + rr
                pltpu.sync_copy(cnt_hbm.at[r], cnt_v)
                cv = cnt_v[pl.ds(0, 16)]
                cnt = jnp.where(core == 0, cv[p], cv[npc + p])
                nb = (cnt + (_EB - 1)) // _EB

                def batch(i, _):
                    off = i * _EB
                    pltpu.sync_copy(keyb_hbm.at[b_idx, r, pl.ds(off, _EB)],
                                    key_v)
                    pltpu.sync_copy(ewb_hbm.at[b_idx, r, pl.ds(off, _EB)],
                                    ew_v)

                    def unpack(j, _):
                        k16 = key_v[pl.ds(j * 16, 16)]
                        idx_v[pl.ds(j * 16, 16)] = k16 & 0xFFFF
                        dloc_v[pl.ds(j * 16, 16)] = k16 >> 16
                        return 0
                    lax.fori_loop(0, _EB // 16, unpack, 0)

                    pltpu.async_copy(pre_hbm.at[idx_v], rows_v, sem).wait()

                    def scale(g, _):
                        w16 = ew_v[pl.ds(g * 16, 16)]
                        for l in range(16):
                            e = g * 16 + l
                            wv = jnp.full((16,), w16[l], jnp.float32)
                            for j in range(nw):
                                rows_v[e, pl.ds(j * 16, 16)] = (
                                    rows_v[e, pl.ds(j * 16, 16)] * wv)
                        return 0
                    lax.fori_loop(0, _EB // 16, scale, 0)

                    pltpu.sync_copy(rows_v, shared.at[dloc_v], add=True)
                    return 0
                lax.fori_loop(0, nb, batch, 0)
            plsc.subcore_barrier()
            pltpu.sync_copy(
                shared.at[pl.ds(sub * rps, rps)],
                agg_hbm.at[pl.ds(b_idx * chunk + sub * rps, rps)])
            plsc.subcore_barrier()

    return seg_kernel


def _gcn_block(x, params, binned, n_pad, e_pad, nch, chunk):
    keyb, ewb, cnts = binned
    acts = [x]
    L = len(params)
    for i, (w0, w1, b) in enumerate(params):
        xi = acts[-1]
        o = w0.shape[1]
        op = max(o, 128)
        osc = o if o >= 16 else 16
        wc = jnp.zeros((xi.shape[1], 2 * op), jnp.float32)
        wc = wc.at[:, :o].set(w0).at[:, op:op + o].set(w1)
        bc = jnp.zeros((1, op), jnp.float32).at[0, :o].set(b[:o])
        dense, pre = _mm2(xi, wc, bc, op)
        pre_sc = pre if osc == op else pre[:, :osc]
        agg = _make_seg_kernel(e_pad, nch, chunk, n_pad, osc)(
            pre_sc, keyb, ewb, cnts)
        if osc != op:
            agg = jnp.pad(agg, ((0, 0), (0, op - osc)))
        h = _finish(dense, agg, acts[-2] if i in (2, 4, 6, 8, 10, 12) else None,
                    relu=(i < L - 1))
        acts.append(h)
    return acts[-1], acts[-2]


def _bin_edges(ei, ew, nch, chunk):
    e = ew.shape[0]
    e_pad = _round_up(e, 512)
    pad = e_pad - e
    src = jnp.pad(ei[0], (0, pad))
    dst = jnp.pad(ei[1], (0, pad))
    eww = jnp.pad(ew, (0, pad))
    return _make_bin_kernel(e_pad, nch, chunk)(src, dst, eww), e_pad


def _cnn(img, params):
    x = img
    feats = []
    for i, (w, b) in enumerate(params):
        s = _CNN_SPECS[i][2]
        x = jax.lax.conv_general_dilated(
            x, w, (s, s), "SAME", dimension_numbers=("NHWC", "HWIO", "NHWC"))
        x = jax.nn.relu(x + b)
        if i in _FEAT_TAPS:
            feats.append(x[0])
    return feats


def _bilinear(im, y, x):
    h, w = im.shape[0], im.shape[1]
    y = jnp.clip(y, 0.0, h - 1.0)
    x = jnp.clip(x, 0.0, w - 1.0)
    y0 = jnp.floor(y).astype(jnp.int32)
    x0 = jnp.floor(x).astype(jnp.int32)
    y1 = jnp.minimum(y0 + 1, h - 1)
    x1 = jnp.minimum(x0 + 1, w - 1)
    wy = (y - y0)[:, None]
    wx = (x - x0)[:, None]
    top = im[y0, x0] * (1.0 - wx) + im[y0, x1] * wx
    bot = im[y1, x0] * (1.0 - wx) + im[y1, x1] * wx
    return top * (1.0 - wy) + bot * wy


def _projection(coords, cameras, feats):
    xc = coords @ cameras[:, :3].T + cameras[:, 3]
    z = xc[:, 2]
    z = jnp.where(jnp.abs(z) < 1e-2, 1e-2, z)
    hh = jnp.clip(250.0 * (-xc[:, 1] / z) + 112.0, 0.0, 223.0)
    ww = jnp.clip(250.0 * (xc[:, 0] / z) + 112.0, 0.0, 223.0)
    outs = [coords]
    for im in feats:
        s = im.shape[0]
        outs.append(_bilinear(im, hh * (s / 224.0), ww * (s / 224.0)))
    return jnp.concatenate(outs, axis=1)


def _pool(x, pidx):
    new = 0.5 * (x[pidx[:, 0]] + x[pidx[:, 1]])
    return jnp.concatenate([x, new], axis=0)


def _pad_rows(x, n_pad):
    return jnp.zeros((n_pad, x.shape[1]), x.dtype).at[:x.shape[0]].set(x)


def kernel(img_all_view, cameras, features, edge_index1, edge_w1, edge_index2,
           edge_w2, edge_index3, edge_w3, pool_idx1, pool_idx2, cnn_params,
           gcn_params):
    n1p, n2p, n3p = 10240, 20480, 40960
    feats = _cnn(img_all_view, cnn_params)
    p1, p2, p3 = gcn_params

    bin1, e1p = _bin_edges(edge_index1, edge_w1, 2, 5120)
    bin2, e2p = _bin_edges(edge_index2, edge_w2, 4, 5120)
    bin3, e3p = _bin_edges(edge_index3, edge_w3, 8, 5120)

    x1 = _pad_rows(_projection(features, cameras, feats), n1p)
    o1, h1 = _gcn_block(x1, p1, bin1, n1p, e1p, 2, 5120)
    out1 = o1[:_N1, :3]
    hid1 = h1[:_N1]

    x2 = jnp.concatenate([_projection(out1, cameras, feats), hid1], axis=1)
    x2 = _pad_rows(_pool(x2, pool_idx1), n2p)
    o2, h2 = _gcn_block(x2, p2, bin2, n2p, e2p, 4, 5120)
    out2 = o2[:_N2, :3]
    hid2 = h2[:_N2]

    x3 = jnp.concatenate([_projection(out2, cameras, feats), hid2], axis=1)
    x3 = _pad_rows(_pool(x3, pool_idx2), n3p)
    o3, _ = _gcn_block(x3, p3, bin3, n3p, e3p, 8, 5120)
    out3 = o3[:_N3, :3]

    out1_2 = _pool(out1, pool_idx1)
    out2_2 = _pool(out2, pool_idx2)
    return (out1, out2, out3, out1_2, out2_2)
